# baseline (speedup 1.0000x reference)
import functools

import jax
import jax.numpy as jnp
from jax import lax
from jax.experimental import pallas as pl
from jax.experimental.pallas import tpu as pltpu
from jax.experimental.pallas import tpu_sc as plsc

N = 10000
R = 8
H = 160
NB = 5
BS = H // NB
E = 320000

NC = 2
NS = 16
NW = NC * NS
LANES = 16

CH = 128
NCH = E // CH
HV = H // LANES

_MESH = plsc.VectorSubcoreMesh(core_axis_name="c", subcore_axis_name="s")

_NITER_P = (NCH + NW - 1) // NW

_CSTRIP = 128
_CROWS = N // NS
_CNS = _CROWS // _CSTRIP
_CTAIL = _CROWS - _CNS * _CSTRIP


@functools.partial(
    pl.kernel,
    out_type=jax.ShapeDtypeStruct((NC * N, 8), jnp.float32),
    mesh=_MESH,
    scratch_types=[
        pltpu.VMEM((CH,), jnp.int32),
        pltpu.VMEM((CH,), jnp.int32),
        pltpu.VMEM((CH,), jnp.int32),
        pltpu.VMEM((CH,), jnp.int32),
        pltpu.VMEM((CH, 8), jnp.float32),
        pltpu.VMEM((CH, 8), jnp.float32),
        pltpu.VMEM((_CSTRIP, 8), jnp.float32),
        pltpu.VMEM_SHARED((N + 8, 8), jnp.float32),
        pltpu.SemaphoreType.DMA,
        pltpu.SemaphoreType.DMA,
        pltpu.SemaphoreType.DMA,
        pltpu.SemaphoreType.DMA,
        pltpu.SemaphoreType.DMA,
        pltpu.SemaphoreType.DMA,
        pltpu.SemaphoreType.DMA,
    ],
    compiler_params=pltpu.CompilerParams(use_tc_tiling_on_sc=False),
)
def _p1(dst_hbm, typ_hbm, eye_hbm, zero_hbm, cnt_hbm,
        d0, d1, t0, t1, v0, v1, zbuf, cnt_sh,
        si0, si1, sg0, sg1, ss0, ss1, sini):
    cid = lax.axis_index("c")
    sid = lax.axis_index("s")
    wid = cid * NS + sid
    D = (d0, d1)
    T = (t0, t1)
    V = (v0, v1)
    SI = (si0, si1)
    SG = (sg0, sg1)
    SS = (ss0, ss1)

    pltpu.async_copy(zero_hbm, zbuf, sini)
    pltpu.make_async_copy(zero_hbm, zbuf, sini).wait()
    zrow = sid * _CROWS

    def zero_strip(i, carry):
        pltpu.sync_copy(zbuf.at[pl.ds(0, _CSTRIP)],
                        cnt_sh.at[pl.ds(zrow + i * _CSTRIP, _CSTRIP)])
        return carry

    lax.fori_loop(0, _CNS, zero_strip, 0)
    pltpu.sync_copy(zbuf.at[pl.ds(0, _CTAIL)],
                    cnt_sh.at[pl.ds(zrow + _CNS * _CSTRIP, _CTAIL)])

    @pl.when(sid == 0)
    def _():
        pltpu.sync_copy(zbuf.at[pl.ds(0, 8)], cnt_sh.at[pl.ds(N, 8)])

    plsc.subcore_barrier()

    def c_of(i):
        c = wid + i * NW
        return jnp.where(c >= NCH, wid, c)

    def is_pad(i):
        return (wid + i * NW) >= NCH

    def issue_idx(s, i):
        base = c_of(i) * CH
        pltpu.async_copy(dst_hbm.at[pl.ds(base, CH)], D[s], SI[s])
        pltpu.async_copy(typ_hbm.at[pl.ds(base, CH)], T[s], SI[s])

    def wait_idx(s):
        pltpu.make_async_copy(dst_hbm.at[pl.ds(0, CH)], D[s], SI[s]).wait()
        pltpu.make_async_copy(typ_hbm.at[pl.ds(0, CH)], T[s], SI[s]).wait()

    def issue_gather(s):
        pltpu.async_copy(eye_hbm.at[T[s]], V[s], SG[s])

    def wait_gather(s):
        pltpu.make_async_copy(eye_hbm.at[T[s]], V[s], SG[s]).wait()

    def issue_scat(s):
        pltpu.async_copy(V[s], cnt_sh.at[D[s]], SS[s], add=True)

    def wait_scat(s):
        pltpu.make_async_copy(V[s], cnt_sh.at[D[s]], SS[s]).wait()

    def pad_fix(s, i):
        @pl.when(is_pad(i))
        def _():
            for v in range(CH // LANES):
                D[s][pl.ds(v * LANES, LANES)] = jnp.full((LANES,), N,
                                                         jnp.int32)

    def half(i, s):
        t = 1 - s
        wait_scat(t)
        issue_idx(t, i + 1)
        wait_gather(s)
        wait_idx(t)
        issue_gather(t)
        pad_fix(s, i)
        issue_scat(s)

    issue_idx(0, 0)
    wait_idx(0)
    issue_gather(0)
    issue_idx(1, 1)
    wait_gather(0)
    wait_idx(1)
    issue_gather(1)
    pad_fix(0, 0)
    issue_scat(0)

    def pair(p, carry):
        half(1 + 2 * p, 1)
        half(2 + 2 * p, 0)
        return carry

    lax.fori_loop(0, (_NITER_P - 1) // 2, pair, 0)
    wait_gather(1)
    wait_scat(0)
    plsc.subcore_barrier()

    def dump(i, carry):
        pltpu.sync_copy(cnt_sh.at[pl.ds(zrow + i * _CSTRIP, _CSTRIP)], zbuf)
        pltpu.sync_copy(
            zbuf, cnt_hbm.at[pl.ds(cid * N + zrow + i * _CSTRIP, _CSTRIP)])
        return carry

    lax.fori_loop(0, _CNS, dump, 0)
    tail = _CNS * _CSTRIP
    pltpu.sync_copy(cnt_sh.at[pl.ds(zrow + tail, _CTAIL)],
                    zbuf.at[pl.ds(0, _CTAIL)])
    pltpu.sync_copy(zbuf.at[pl.ds(0, _CTAIL)],
                    cnt_hbm.at[pl.ds(cid * N + zrow + tail, _CTAIL)])


def _p2_body(cnt_ref, inv_ref):
    c = cnt_ref[0] + cnt_ref[1]
    inv_ref[...] = 1.0 / jnp.maximum(c, 1.0)


def _p2(cnt):
    cnt3 = cnt.reshape(NC, (N * 8) // 128, 128)
    inv = pl.pallas_call(
        _p2_body,
        out_shape=jax.ShapeDtypeStruct(((N * 8) // 128, 128), jnp.float32),
    )(cnt3)
    return inv.reshape(N, 8)


_TM = 1000


def _t_body(x_ref, w_ref, b_ref, out_ref):
    g = pl.program_id(0)
    y = jnp.dot(x_ref[...], w_ref[0], preferred_element_type=jnp.float32)
    is_root = (g == R).astype(jnp.float32)
    out_ref[0] = y + b_ref[...] * is_root


def _t(x, wall, b):
    return pl.pallas_call(
        _t_body,
        grid=(R + 1, N // _TM),
        in_specs=[
            pl.BlockSpec((_TM, H), lambda g, i: (i, 0)),
            pl.BlockSpec((1, H, H), lambda g, i: (g, 0, 0)),
            pl.BlockSpec((1, H), lambda g, i: (0, 0)),
        ],
        out_specs=pl.BlockSpec((1, _TM, H), lambda g, i: (g, i, 0)),
        out_shape=jax.ShapeDtypeStruct((R + 1, N, H), jnp.float32),
    )(x, wall, b.reshape(1, H))


SCH = 80
NCH_S = E // SCH
_NITER = (NCH_S + NW - 1) // NW
_NSTRIP = 1000 // SCH
_TAIL = 1000 - _NSTRIP * SCH


@functools.partial(
    pl.kernel,
    out_type=jax.ShapeDtypeStruct((NC, N, H), jnp.float32),
    mesh=_MESH,
    scratch_types=[
        pltpu.VMEM((SCH,), jnp.int32),
        pltpu.VMEM((SCH,), jnp.int32),
        pltpu.VMEM((SCH,), jnp.int32),
        pltpu.VMEM((SCH,), jnp.int32),
        pltpu.VMEM((SCH,), jnp.int32),
        pltpu.VMEM((SCH,), jnp.int32),
        pltpu.VMEM((SCH,), jnp.int32),
        pltpu.VMEM((SCH,), jnp.int32),
        pltpu.VMEM((SCH, 8), jnp.float32),
        pltpu.VMEM((SCH, 8), jnp.float32),
        pltpu.VMEM((SCH, H), jnp.float32),
        pltpu.VMEM((SCH, H), jnp.float32),
        pltpu.VMEM_SHARED((N + 8, H), jnp.float32),
        pltpu.SemaphoreType.DMA,
        pltpu.SemaphoreType.DMA,
        pltpu.SemaphoreType.DMA,
        pltpu.SemaphoreType.DMA,
        pltpu.SemaphoreType.DMA,
        pltpu.SemaphoreType.DMA,
    ],
    compiler_params=pltpu.CompilerParams(needs_layout_passes=False,
                                         use_tc_tiling_on_sc=False),
)
def _s(ytab_hbm, src_hbm, dst_hbm, typ_hbm, inv_hbm, part_hbm,
       s0, s1, d0, d1, t0, t1, g0, g1, n0, n1, r0, r1, acc,
       si0, si1, sg0, sg1, ss0, ss1):
    cid = lax.axis_index("c")
    sid = lax.axis_index("s")
    wid = cid * NS + sid
    S_ = (s0, s1)
    D = (d0, d1)
    T = (t0, t1)
    G = (g0, g1)
    NM = (n0, n1)
    RW = (r0, r1)
    SI = (si0, si1)
    SG = (sg0, sg1)
    SS = (ss0, ss1)

    def zrow(r, carry):
        for k in range(HV):
            r0[r, pl.ds(k * LANES, LANES)] = jnp.zeros((LANES,), jnp.float32)
        return carry

    lax.fori_loop(0, SCH, zrow, 0)
    arow = sid * 1000

    @pl.when(sid < 10)
    def _():
        def zcopy(i, carry):
            pltpu.sync_copy(r0.at[pl.ds(0, SCH)],
                            acc.at[pl.ds(arow + i * SCH, SCH)])
            return carry

        lax.fori_loop(0, _NSTRIP, zcopy, 0)
        pltpu.sync_copy(r0.at[pl.ds(0, _TAIL)],
                        acc.at[pl.ds(arow + _NSTRIP * SCH, _TAIL)])

    @pl.when(sid == 10)
    def _():
        pltpu.sync_copy(r0.at[pl.ds(0, 8)], acc.at[pl.ds(N, 8)])

    plsc.subcore_barrier()

    def c_of(i):
        c = wid + i * NW
        return jnp.where(c >= NCH_S, wid, c)

    def is_pad(i):
        return (wid + i * NW) >= NCH_S

    def issue_idx(s, i):
        base = c_of(i) * SCH
        pltpu.async_copy(src_hbm.at[pl.ds(base, SCH)], S_[s], SI[s])
        pltpu.async_copy(dst_hbm.at[pl.ds(base, SCH)], D[s], SI[s])
        pltpu.async_copy(typ_hbm.at[pl.ds(base, SCH)], T[s], SI[s])

    def wait_idx(s):
        pltpu.make_async_copy(src_hbm.at[pl.ds(0, SCH)], S_[s], SI[s]).wait()
        pltpu.make_async_copy(dst_hbm.at[pl.ds(0, SCH)], D[s], SI[s]).wait()
        pltpu.make_async_copy(typ_hbm.at[pl.ds(0, SCH)], T[s], SI[s]).wait()

    def compute_idx(s):
        def vec(v, carry):
            sl = pl.ds(v * LANES, LANES)
            G[s][sl] = T[s][sl] * N + S_[s][sl]
            return carry

        lax.fori_loop(0, SCH // LANES, vec, 0)

    def issue_gather(s):
        pltpu.async_copy(ytab_hbm.at[G[s]], RW[s], SG[s])
        pltpu.async_copy(inv_hbm.at[D[s]], NM[s], SG[s])

    def wait_gather(s):
        pltpu.make_async_copy(ytab_hbm.at[G[s]], RW[s], SG[s]).wait()
        pltpu.make_async_copy(inv_hbm.at[D[s]], NM[s], SG[s]).wait()

    def issue_scat(s):
        pltpu.async_copy(RW[s], acc.at[D[s]], SS[s], add=True)

    def wait_scat(s):
        pltpu.make_async_copy(RW[s], acc.at[D[s]], SS[s]).wait()

    def scale(s, i):
        def body(e, carry):
            ev = jnp.broadcast_to(e, (LANES,))
            tv = plsc.load_gather(T[s], [ev])
            nv = plsc.load_gather(NM[s], [ev, tv])
            for k in range(HV):
                sl = pl.ds(k * LANES, LANES)
                RW[s][e, sl] = RW[s][e, sl] * nv
            return carry

        lax.fori_loop(0, SCH, body, 0)

        @pl.when(is_pad(i))
        def _():
            for v in range(SCH // LANES):
                D[s][pl.ds(v * LANES, LANES)] = jnp.full((LANES,), N,
                                                         jnp.int32)

    def half(i, s):
        t = 1 - s
        wait_scat(t)
        issue_idx(t, i + 1)
        wait_gather(s)
        wait_idx(t)
        compute_idx(t)
        issue_gather(t)
        scale(s, i)
        issue_scat(s)

    issue_idx(0, 0)
    wait_idx(0)
    compute_idx(0)
    issue_gather(0)
    issue_idx(1, 1)
    wait_gather(0)
    wait_idx(1)
    compute_idx(1)
    issue_gather(1)
    scale(0, 0)
    issue_scat(0)

    def pair(p, carry):
        half(1 + 2 * p, 1)
        half(2 + 2 * p, 0)
        return carry

    lax.fori_loop(0, (_NITER - 1) // 2, pair, 0)
    wait_gather(1)
    wait_scat(0)

    plsc.subcore_barrier()

    @pl.when(sid < 10)
    def _():
        def dump(i, carry):
            pltpu.sync_copy(acc.at[pl.ds(arow + i * SCH, SCH)], r0)
            pltpu.sync_copy(r0, part_hbm.at[cid, pl.ds(arow + i * SCH, SCH)])
            return carry

        lax.fori_loop(0, _NSTRIP, dump, 0)
        pltpu.sync_copy(acc.at[pl.ds(arow + _NSTRIP * SCH, _TAIL)],
                        r0.at[pl.ds(0, _TAIL)])
        pltpu.sync_copy(r0.at[pl.ds(0, _TAIL)],
                        part_hbm.at[cid, pl.ds(arow + _NSTRIP * SCH, _TAIL)])


def _c_body(relu, p_ref, z_ref, out_ref):
    y = p_ref[0] + p_ref[1] + z_ref[...]
    if relu:
        y = jnp.maximum(y, 0.0)
    out_ref[...] = y


def _c(parts, z, relu):
    return pl.pallas_call(
        functools.partial(_c_body, relu),
        grid=(N // _TM,),
        in_specs=[
            pl.BlockSpec((NC, _TM, H), lambda i: (0, i, 0)),
            pl.BlockSpec((_TM, H), lambda i: (i, 0)),
        ],
        out_specs=pl.BlockSpec((_TM, H), lambda i: (i, 0)),
        out_shape=jax.ShapeDtypeStruct((N, H), jnp.float32),
    )(parts, z)


def _block_diag_weights(w, root):
    wd = jnp.zeros((R, NB, BS, NB, BS), jnp.float32)
    idx = jnp.arange(NB)
    wd = wd.at[:, idx, :, idx, :].set(w.transpose(1, 0, 2, 3))
    wd = wd.reshape(R, H, H)
    return jnp.concatenate([wd, root[None]], axis=0)


def kernel(node_emb, w0, root0, b0, w1, root1, b1, edge_index, edge_type):
    src = edge_index[0]
    dst = edge_index[1]
    eye8 = jnp.eye(8, dtype=jnp.float32)
    zeros8 = jnp.zeros((_CSTRIP, 8), jnp.float32)
    cnt = _p1(dst, edge_type, eye8, zeros8)
    inv = _p2(cnt)

    wall0 = _block_diag_weights(w0, root0)
    wall1 = _block_diag_weights(w1, root1)

    x = node_emb
    for wall, b, relu in ((wall0, b0, True), (wall1, b1, True),
                          (wall1, b1, False)):
        ytab = _t(x, wall, b)
        parts = _s(ytab.reshape((R + 1) * N, H), src, dst, edge_type, inv)
        x = _c(parts, ytab[R].reshape(N, H), relu)
    return x


# speedup vs baseline: 1.0625x; 1.0625x over previous
import functools

import jax
import jax.numpy as jnp
from jax import lax
from jax.experimental import pallas as pl
from jax.experimental.pallas import tpu as pltpu
from jax.experimental.pallas import tpu_sc as plsc

N = 10000
R = 8
H = 160
NB = 5
BS = H // NB
E = 320000

NC = 2
NS = 16
NW = NC * NS
LANES = 16

CH = 128
NCH = E // CH
HV = H // LANES

_MESH = plsc.VectorSubcoreMesh(core_axis_name="c", subcore_axis_name="s")

_NITER_P = (NCH + NW - 1) // NW

_CSTRIP = 128
_CROWS = N // NS
_CNS = _CROWS // _CSTRIP
_CTAIL = _CROWS - _CNS * _CSTRIP


@functools.partial(
    pl.kernel,
    out_type=jax.ShapeDtypeStruct((NC * N, 8), jnp.float32),
    mesh=_MESH,
    scratch_types=[
        pltpu.VMEM((CH,), jnp.int32),
        pltpu.VMEM((CH,), jnp.int32),
        pltpu.VMEM((CH,), jnp.int32),
        pltpu.VMEM((CH,), jnp.int32),
        pltpu.VMEM((CH, 8), jnp.float32),
        pltpu.VMEM((CH, 8), jnp.float32),
        pltpu.VMEM((_CSTRIP, 8), jnp.float32),
        pltpu.VMEM_SHARED((N + 8, 8), jnp.float32),
        pltpu.SemaphoreType.DMA,
        pltpu.SemaphoreType.DMA,
        pltpu.SemaphoreType.DMA,
        pltpu.SemaphoreType.DMA,
        pltpu.SemaphoreType.DMA,
        pltpu.SemaphoreType.DMA,
        pltpu.SemaphoreType.DMA,
    ],
    compiler_params=pltpu.CompilerParams(use_tc_tiling_on_sc=False),
)
def _p1(dst_hbm, typ_hbm, eye_hbm, zero_hbm, cnt_hbm,
        d0, d1, t0, t1, v0, v1, zbuf, cnt_sh,
        si0, si1, sg0, sg1, ss0, ss1, sini):
    cid = lax.axis_index("c")
    sid = lax.axis_index("s")
    wid = cid * NS + sid
    D = (d0, d1)
    T = (t0, t1)
    V = (v0, v1)
    SI = (si0, si1)
    SG = (sg0, sg1)
    SS = (ss0, ss1)

    pltpu.async_copy(zero_hbm, zbuf, sini)
    pltpu.make_async_copy(zero_hbm, zbuf, sini).wait()
    zrow = sid * _CROWS

    def zero_strip(i, carry):
        pltpu.sync_copy(zbuf.at[pl.ds(0, _CSTRIP)],
                        cnt_sh.at[pl.ds(zrow + i * _CSTRIP, _CSTRIP)])
        return carry

    lax.fori_loop(0, _CNS, zero_strip, 0)
    pltpu.sync_copy(zbuf.at[pl.ds(0, _CTAIL)],
                    cnt_sh.at[pl.ds(zrow + _CNS * _CSTRIP, _CTAIL)])

    @pl.when(sid == 0)
    def _():
        pltpu.sync_copy(zbuf.at[pl.ds(0, 8)], cnt_sh.at[pl.ds(N, 8)])

    plsc.subcore_barrier()

    def c_of(i):
        c = wid + i * NW
        return jnp.where(c >= NCH, wid, c)

    def is_pad(i):
        return (wid + i * NW) >= NCH

    def issue_idx(s, i):
        base = c_of(i) * CH
        pltpu.async_copy(dst_hbm.at[pl.ds(base, CH)], D[s], SI[s])
        pltpu.async_copy(typ_hbm.at[pl.ds(base, CH)], T[s], SI[s])

    def wait_idx(s):
        pltpu.make_async_copy(dst_hbm.at[pl.ds(0, CH)], D[s], SI[s]).wait()
        pltpu.make_async_copy(typ_hbm.at[pl.ds(0, CH)], T[s], SI[s]).wait()

    def issue_gather(s):
        pltpu.async_copy(eye_hbm.at[T[s]], V[s], SG[s])

    def wait_gather(s):
        pltpu.make_async_copy(eye_hbm.at[T[s]], V[s], SG[s]).wait()

    def issue_scat(s):
        pltpu.async_copy(V[s], cnt_sh.at[D[s]], SS[s], add=True)

    def wait_scat(s):
        pltpu.make_async_copy(V[s], cnt_sh.at[D[s]], SS[s]).wait()

    def pad_fix(s, i):
        @pl.when(is_pad(i))
        def _():
            for v in range(CH // LANES):
                D[s][pl.ds(v * LANES, LANES)] = jnp.full((LANES,), N,
                                                         jnp.int32)

    def half(i, s):
        t = 1 - s
        wait_scat(t)
        issue_idx(t, i + 1)
        wait_gather(s)
        wait_idx(t)
        issue_gather(t)
        pad_fix(s, i)
        issue_scat(s)

    issue_idx(0, 0)
    wait_idx(0)
    issue_gather(0)
    issue_idx(1, 1)
    wait_gather(0)
    wait_idx(1)
    issue_gather(1)
    pad_fix(0, 0)
    issue_scat(0)

    def pair(p, carry):
        half(1 + 2 * p, 1)
        half(2 + 2 * p, 0)
        return carry

    lax.fori_loop(0, (_NITER_P - 1) // 2, pair, 0)
    wait_gather(1)
    wait_scat(0)
    plsc.subcore_barrier()

    def dump(i, carry):
        pltpu.sync_copy(cnt_sh.at[pl.ds(zrow + i * _CSTRIP, _CSTRIP)], zbuf)
        pltpu.sync_copy(
            zbuf, cnt_hbm.at[pl.ds(cid * N + zrow + i * _CSTRIP, _CSTRIP)])
        return carry

    lax.fori_loop(0, _CNS, dump, 0)
    tail = _CNS * _CSTRIP
    pltpu.sync_copy(cnt_sh.at[pl.ds(zrow + tail, _CTAIL)],
                    zbuf.at[pl.ds(0, _CTAIL)])
    pltpu.sync_copy(zbuf.at[pl.ds(0, _CTAIL)],
                    cnt_hbm.at[pl.ds(cid * N + zrow + tail, _CTAIL)])


def _p2_body(cnt_ref, inv_ref):
    c = cnt_ref[0] + cnt_ref[1]
    inv_ref[...] = 1.0 / jnp.maximum(c, 1.0)


def _p2(cnt):
    cnt3 = cnt.reshape(NC, (N * 8) // 128, 128)
    inv = pl.pallas_call(
        _p2_body,
        out_shape=jax.ShapeDtypeStruct(((N * 8) // 128, 128), jnp.float32),
    )(cnt3)
    return inv.reshape(N, 8)


_TM = 1000


def _t_body(x_ref, w_ref, b_ref, out_ref):
    g = pl.program_id(0)
    y = jnp.dot(x_ref[...], w_ref[0], preferred_element_type=jnp.float32)
    is_root = (g == R).astype(jnp.float32)
    out_ref[0] = y + b_ref[...] * is_root


def _t(x, wall, b):
    return pl.pallas_call(
        _t_body,
        grid=(R + 1, N // _TM),
        in_specs=[
            pl.BlockSpec((_TM, H), lambda g, i: (i, 0)),
            pl.BlockSpec((1, H, H), lambda g, i: (g, 0, 0)),
            pl.BlockSpec((1, H), lambda g, i: (0, 0)),
        ],
        out_specs=pl.BlockSpec((1, _TM, H), lambda g, i: (g, i, 0)),
        out_shape=jax.ShapeDtypeStruct((R + 1, N, H), jnp.float32),
    )(x, wall, b.reshape(1, H))


SCH = 80
NCH_S = E // SCH
_NITER = (NCH_S + NW - 1) // NW
_NSTRIP = 1000 // SCH
_TAIL = 1000 - _NSTRIP * SCH


@functools.partial(
    pl.kernel,
    out_type=jax.ShapeDtypeStruct((NC, N, H), jnp.float32),
    mesh=_MESH,
    scratch_types=[
        pltpu.VMEM((SCH,), jnp.int32),
        pltpu.VMEM((SCH,), jnp.int32),
        pltpu.VMEM((SCH,), jnp.int32),
        pltpu.VMEM((SCH,), jnp.int32),
        pltpu.VMEM((SCH,), jnp.int32),
        pltpu.VMEM((SCH,), jnp.int32),
        pltpu.VMEM((SCH,), jnp.int32),
        pltpu.VMEM((SCH,), jnp.int32),
        pltpu.VMEM((SCH, 8), jnp.float32),
        pltpu.VMEM((SCH, 8), jnp.float32),
        pltpu.VMEM((SCH,), jnp.float32),
        pltpu.VMEM((SCH,), jnp.float32),
        pltpu.VMEM((SCH, H), jnp.float32),
        pltpu.VMEM((SCH, H), jnp.float32),
        pltpu.VMEM_SHARED((N + 8, H), jnp.float32),
        pltpu.SemaphoreType.DMA,
        pltpu.SemaphoreType.DMA,
        pltpu.SemaphoreType.DMA,
        pltpu.SemaphoreType.DMA,
        pltpu.SemaphoreType.DMA,
        pltpu.SemaphoreType.DMA,
    ],
    compiler_params=pltpu.CompilerParams(needs_layout_passes=False,
                                         use_tc_tiling_on_sc=False),
)
def _s(ytab_hbm, src_hbm, dst_hbm, typ_hbm, inv_hbm, part_hbm,
       s0, s1, d0, d1, t0, t1, g0, g1, n0, n1, m0, m1, r0, r1, acc,
       si0, si1, sg0, sg1, ss0, ss1):
    cid = lax.axis_index("c")
    sid = lax.axis_index("s")
    wid = cid * NS + sid
    S_ = (s0, s1)
    D = (d0, d1)
    T = (t0, t1)
    G = (g0, g1)
    NM = (n0, n1)
    NMS = (m0, m1)
    RW = (r0, r1)
    SI = (si0, si1)
    SG = (sg0, sg1)
    SS = (ss0, ss1)

    def zrow(r, carry):
        for k in range(HV):
            r0[r, pl.ds(k * LANES, LANES)] = jnp.zeros((LANES,), jnp.float32)
        return carry

    lax.fori_loop(0, SCH, zrow, 0)
    arow = sid * 1000

    @pl.when(sid < 10)
    def _():
        def zcopy(i, carry):
            pltpu.sync_copy(r0.at[pl.ds(0, SCH)],
                            acc.at[pl.ds(arow + i * SCH, SCH)])
            return carry

        lax.fori_loop(0, _NSTRIP, zcopy, 0)
        pltpu.sync_copy(r0.at[pl.ds(0, _TAIL)],
                        acc.at[pl.ds(arow + _NSTRIP * SCH, _TAIL)])

    @pl.when(sid == 10)
    def _():
        pltpu.sync_copy(r0.at[pl.ds(0, 8)], acc.at[pl.ds(N, 8)])

    plsc.subcore_barrier()

    def c_of(i):
        c = wid + i * NW
        return jnp.where(c >= NCH_S, wid, c)

    def is_pad(i):
        return (wid + i * NW) >= NCH_S

    def issue_idx(s, i):
        base = c_of(i) * SCH
        pltpu.async_copy(src_hbm.at[pl.ds(base, SCH)], S_[s], SI[s])
        pltpu.async_copy(dst_hbm.at[pl.ds(base, SCH)], D[s], SI[s])
        pltpu.async_copy(typ_hbm.at[pl.ds(base, SCH)], T[s], SI[s])

    def wait_idx(s):
        pltpu.make_async_copy(src_hbm.at[pl.ds(0, SCH)], S_[s], SI[s]).wait()
        pltpu.make_async_copy(dst_hbm.at[pl.ds(0, SCH)], D[s], SI[s]).wait()
        pltpu.make_async_copy(typ_hbm.at[pl.ds(0, SCH)], T[s], SI[s]).wait()

    def compute_idx(s):
        def vec(v, carry):
            sl = pl.ds(v * LANES, LANES)
            G[s][sl] = T[s][sl] * N + S_[s][sl]
            return carry

        lax.fori_loop(0, SCH // LANES, vec, 0)

    def issue_gather(s):
        pltpu.async_copy(ytab_hbm.at[G[s]], RW[s], SG[s])
        pltpu.async_copy(inv_hbm.at[D[s]], NM[s], SG[s])

    def wait_gather(s):
        pltpu.make_async_copy(ytab_hbm.at[G[s]], RW[s], SG[s]).wait()
        pltpu.make_async_copy(inv_hbm.at[D[s]], NM[s], SG[s]).wait()

    def issue_scat(s):
        pltpu.async_copy(RW[s], acc.at[D[s]], SS[s], add=True)

    def wait_scat(s):
        pltpu.make_async_copy(RW[s], acc.at[D[s]], SS[s]).wait()

    def scale(s, i):
        lanes_iota = lax.iota(jnp.int32, LANES)
        for v in range(SCH // LANES):
            sl = pl.ds(v * LANES, LANES)
            nv16 = plsc.load_gather(NM[s], [lanes_iota + v * LANES, T[s][sl]])
            NMS[s][sl] = nv16

        def body(e, carry):
            nv = plsc.load_gather(NMS[s], [jnp.broadcast_to(e, (LANES,))])
            for k in range(HV):
                sl = pl.ds(k * LANES, LANES)
                RW[s][e, sl] = RW[s][e, sl] * nv
            return carry

        lax.fori_loop(0, SCH, body, 0)

        @pl.when(is_pad(i))
        def _():
            for v in range(SCH // LANES):
                D[s][pl.ds(v * LANES, LANES)] = jnp.full((LANES,), N,
                                                         jnp.int32)

    def half(i, s):
        t = 1 - s
        wait_scat(t)
        issue_idx(t, i + 1)
        wait_gather(s)
        wait_idx(t)
        compute_idx(t)
        issue_gather(t)
        scale(s, i)
        issue_scat(s)

    issue_idx(0, 0)
    wait_idx(0)
    compute_idx(0)
    issue_gather(0)
    issue_idx(1, 1)
    wait_gather(0)
    wait_idx(1)
    compute_idx(1)
    issue_gather(1)
    scale(0, 0)
    issue_scat(0)

    def pair(p, carry):
        half(1 + 2 * p, 1)
        half(2 + 2 * p, 0)
        return carry

    lax.fori_loop(0, (_NITER - 1) // 2, pair, 0)
    wait_gather(1)
    wait_scat(0)

    plsc.subcore_barrier()

    @pl.when(sid < 10)
    def _():
        def dump(i, carry):
            pltpu.sync_copy(acc.at[pl.ds(arow + i * SCH, SCH)], r0)
            pltpu.sync_copy(r0, part_hbm.at[cid, pl.ds(arow + i * SCH, SCH)])
            return carry

        lax.fori_loop(0, _NSTRIP, dump, 0)
        pltpu.sync_copy(acc.at[pl.ds(arow + _NSTRIP * SCH, _TAIL)],
                        r0.at[pl.ds(0, _TAIL)])
        pltpu.sync_copy(r0.at[pl.ds(0, _TAIL)],
                        part_hbm.at[cid, pl.ds(arow + _NSTRIP * SCH, _TAIL)])


def _c_body(relu, p_ref, z_ref, out_ref):
    y = p_ref[0] + p_ref[1] + z_ref[...]
    if relu:
        y = jnp.maximum(y, 0.0)
    out_ref[...] = y


def _c(parts, z, relu):
    return pl.pallas_call(
        functools.partial(_c_body, relu),
        grid=(N // _TM,),
        in_specs=[
            pl.BlockSpec((NC, _TM, H), lambda i: (0, i, 0)),
            pl.BlockSpec((_TM, H), lambda i: (i, 0)),
        ],
        out_specs=pl.BlockSpec((_TM, H), lambda i: (i, 0)),
        out_shape=jax.ShapeDtypeStruct((N, H), jnp.float32),
    )(parts, z)


def _block_diag_weights(w, root):
    wd = jnp.zeros((R, NB, BS, NB, BS), jnp.float32)
    idx = jnp.arange(NB)
    wd = wd.at[:, idx, :, idx, :].set(w.transpose(1, 0, 2, 3))
    wd = wd.reshape(R, H, H)
    return jnp.concatenate([wd, root[None]], axis=0)


def kernel(node_emb, w0, root0, b0, w1, root1, b1, edge_index, edge_type):
    src = edge_index[0]
    dst = edge_index[1]
    eye8 = jnp.eye(8, dtype=jnp.float32)
    zeros8 = jnp.zeros((_CSTRIP, 8), jnp.float32)
    cnt = _p1(dst, edge_type, eye8, zeros8)
    inv = _p2(cnt)

    wall0 = _block_diag_weights(w0, root0)
    wall1 = _block_diag_weights(w1, root1)

    x = node_emb
    for wall, b, relu in ((wall0, b0, True), (wall1, b1, True),
                          (wall1, b1, False)):
        ytab = _t(x, wall, b)
        parts = _s(ytab.reshape((R + 1) * N, H), src, dst, edge_type, inv)
        x = _c(parts, ytab[R].reshape(N, H), relu)
    return x


# speedup vs baseline: 2.0538x; 1.9330x over previous
import functools

import jax
import jax.numpy as jnp
from jax import lax
from jax.experimental import pallas as pl
from jax.experimental.pallas import tpu as pltpu
from jax.experimental.pallas import tpu_sc as plsc

N = 10000
R = 8
H = 160
NB = 5
BS = H // NB
E = 320000

NC = 2
NS = 16
NW = NC * NS
LANES = 16

CH = 128
NCH = E // CH
HV = H // LANES

_MESH = plsc.VectorSubcoreMesh(core_axis_name="c", subcore_axis_name="s")

_NITER_P = (NCH + NW - 1) // NW

NR = N * R
_CSTRIP = 128
_CROWS = NR // NS
_CNS = _CROWS // _CSTRIP
_CTAIL = _CROWS - _CNS * _CSTRIP


@functools.partial(
    pl.kernel,
    out_type=jax.ShapeDtypeStruct((NC * NR, 8), jnp.float32),
    mesh=_MESH,
    scratch_types=[
        pltpu.VMEM((CH,), jnp.int32),
        pltpu.VMEM((CH,), jnp.int32),
        pltpu.VMEM((CH,), jnp.int32),
        pltpu.VMEM((CH,), jnp.int32),
        pltpu.VMEM((CH,), jnp.int32),
        pltpu.VMEM((CH,), jnp.int32),
        pltpu.VMEM((CH, 8), jnp.float32),
        pltpu.VMEM((CH, 8), jnp.float32),
        pltpu.VMEM((_CSTRIP, 8), jnp.float32),
        pltpu.VMEM_SHARED((NR + 8, 8), jnp.float32),
        pltpu.SemaphoreType.DMA,
        pltpu.SemaphoreType.DMA,
        pltpu.SemaphoreType.DMA,
        pltpu.SemaphoreType.DMA,
        pltpu.SemaphoreType.DMA,
    ],
    compiler_params=pltpu.CompilerParams(use_tc_tiling_on_sc=False),
)
def _p1(dst_hbm, typ_hbm, ones_hbm, zero_hbm, cnt_hbm,
        d0, d1, t0, t1, c0, c1, v0, v1, zbuf, cnt_sh,
        si0, si1, ss0, ss1, sini):
    cid = lax.axis_index("c")
    sid = lax.axis_index("s")
    wid = cid * NS + sid
    D = (d0, d1)
    T = (t0, t1)
    CB = (c0, c1)
    V = (v0, v1)
    SI = (si0, si1)
    SS = (ss0, ss1)

    pltpu.async_copy(ones_hbm, v0, sini)
    pltpu.async_copy(ones_hbm, v1, sini)
    pltpu.async_copy(zero_hbm, zbuf, sini)
    pltpu.make_async_copy(ones_hbm, v0, sini).wait()
    pltpu.make_async_copy(ones_hbm, v1, sini).wait()
    pltpu.make_async_copy(zero_hbm, zbuf, sini).wait()
    zrow = sid * _CROWS

    def zero_strip(i, carry):
        pltpu.sync_copy(zbuf.at[pl.ds(0, _CSTRIP)],
                        cnt_sh.at[pl.ds(zrow + i * _CSTRIP, _CSTRIP)])
        return carry

    lax.fori_loop(0, _CNS, zero_strip, 0)
    pltpu.sync_copy(zbuf.at[pl.ds(0, _CTAIL)],
                    cnt_sh.at[pl.ds(zrow + _CNS * _CSTRIP, _CTAIL)])

    @pl.when(sid == 0)
    def _():
        pltpu.sync_copy(zbuf.at[pl.ds(0, 8)], cnt_sh.at[pl.ds(NR, 8)])

    plsc.subcore_barrier()

    def c_of(i):
        c = wid + i * NW
        return jnp.where(c >= NCH, wid, c)

    def is_pad(i):
        return (wid + i * NW) >= NCH

    def issue_idx(s, i):
        base = c_of(i) * CH
        pltpu.async_copy(dst_hbm.at[pl.ds(base, CH)], D[s], SI[s])
        pltpu.async_copy(typ_hbm.at[pl.ds(base, CH)], T[s], SI[s])

    def wait_idx(s):
        pltpu.make_async_copy(dst_hbm.at[pl.ds(0, CH)], D[s], SI[s]).wait()
        pltpu.make_async_copy(typ_hbm.at[pl.ds(0, CH)], T[s], SI[s]).wait()

    def compute(s, i):
        def vec(v, carry):
            sl = pl.ds(v * LANES, LANES)
            CB[s][sl] = D[s][sl] * R + T[s][sl]
            return carry

        lax.fori_loop(0, CH // LANES, vec, 0)

        @pl.when(is_pad(i))
        def _():
            for v in range(CH // LANES):
                CB[s][pl.ds(v * LANES, LANES)] = jnp.full((LANES,), NR,
                                                          jnp.int32)

    def issue_scat(s):
        pltpu.async_copy(V[s], cnt_sh.at[CB[s]], SS[s], add=True)

    def wait_scat(s):
        pltpu.make_async_copy(V[s], cnt_sh.at[CB[s]], SS[s]).wait()

    def half(i, s):
        t = 1 - s
        wait_scat(t)
        issue_idx(t, i + 1)
        wait_idx(s)
        compute(s, i)
        issue_scat(s)

    issue_idx(0, 0)
    issue_idx(1, 1)
    wait_idx(0)
    compute(0, 0)
    issue_scat(0)

    def pair(p, carry):
        half(1 + 2 * p, 1)
        half(2 + 2 * p, 0)
        return carry

    lax.fori_loop(0, (_NITER_P - 1) // 2, pair, 0)
    wait_idx(1)
    wait_scat(0)
    plsc.subcore_barrier()

    def dump(i, carry):
        pltpu.sync_copy(cnt_sh.at[pl.ds(zrow + i * _CSTRIP, _CSTRIP)], zbuf)
        pltpu.sync_copy(
            zbuf, cnt_hbm.at[pl.ds(cid * NR + zrow + i * _CSTRIP, _CSTRIP)])
        return carry

    lax.fori_loop(0, _CNS, dump, 0)
    tail = _CNS * _CSTRIP
    pltpu.sync_copy(cnt_sh.at[pl.ds(zrow + tail, _CTAIL)],
                    zbuf.at[pl.ds(0, _CTAIL)])
    pltpu.sync_copy(zbuf.at[pl.ds(0, _CTAIL)],
                    cnt_hbm.at[pl.ds(cid * NR + zrow + tail, _CTAIL)])


def _p2_body(cnt_ref, inv_ref):
    c = cnt_ref[0] + cnt_ref[1]
    inv_ref[...] = 1.0 / jnp.maximum(c, 1.0)


def _p2(cnt):
    cnt3 = cnt.reshape(NC, (NR * 8) // 128, 128)
    inv = pl.pallas_call(
        _p2_body,
        out_shape=jax.ShapeDtypeStruct(((NR * 8) // 128, 128), jnp.float32),
    )(cnt3)
    return inv.reshape(NR, 8)


_TM = 1000


def _t_body(x_ref, w_ref, b_ref, out_ref):
    g = pl.program_id(0)
    y = jnp.dot(x_ref[...], w_ref[0], preferred_element_type=jnp.float32)
    is_root = (g == R).astype(jnp.float32)
    out_ref[0] = y + b_ref[...] * is_root


def _t(x, wall, b):
    return pl.pallas_call(
        _t_body,
        grid=(R + 1, N // _TM),
        in_specs=[
            pl.BlockSpec((_TM, H), lambda g, i: (i, 0)),
            pl.BlockSpec((1, H, H), lambda g, i: (g, 0, 0)),
            pl.BlockSpec((1, H), lambda g, i: (0, 0)),
        ],
        out_specs=pl.BlockSpec((1, _TM, H), lambda g, i: (g, i, 0)),
        out_shape=jax.ShapeDtypeStruct((R + 1, N, H), jnp.float32),
    )(x, wall, b.reshape(1, H))


SCH = 80
NCH_S = E // SCH
_NITER = (NCH_S + NW - 1) // NW
_NSTRIP = 1000 // SCH
_TAIL = 1000 - _NSTRIP * SCH


@functools.partial(
    pl.kernel,
    out_type=jax.ShapeDtypeStruct((NC, N, H), jnp.float32),
    mesh=_MESH,
    scratch_types=[
        pltpu.VMEM((SCH,), jnp.int32),
        pltpu.VMEM((SCH,), jnp.int32),
        pltpu.VMEM((SCH,), jnp.int32),
        pltpu.VMEM((SCH,), jnp.int32),
        pltpu.VMEM((SCH,), jnp.int32),
        pltpu.VMEM((SCH,), jnp.int32),
        pltpu.VMEM((SCH,), jnp.int32),
        pltpu.VMEM((SCH,), jnp.int32),
        pltpu.VMEM((SCH,), jnp.int32),
        pltpu.VMEM((SCH,), jnp.int32),
        pltpu.VMEM((SCH, 8), jnp.float32),
        pltpu.VMEM((SCH, 8), jnp.float32),
        pltpu.VMEM((SCH, H), jnp.float32),
        pltpu.VMEM((SCH, H), jnp.float32),
        pltpu.VMEM_SHARED((N + 8, H), jnp.float32),
        pltpu.SemaphoreType.DMA,
        pltpu.SemaphoreType.DMA,
        pltpu.SemaphoreType.DMA,
        pltpu.SemaphoreType.DMA,
        pltpu.SemaphoreType.DMA,
        pltpu.SemaphoreType.DMA,
    ],
    compiler_params=pltpu.CompilerParams(needs_layout_passes=False,
                                         use_tc_tiling_on_sc=False),
)
def _s(ytab_hbm, src_hbm, dst_hbm, typ_hbm, inv_hbm, part_hbm,
       s0, s1, d0, d1, t0, t1, g0, g1, cb0, cb1, n0, n1, r0, r1, acc,
       si0, si1, sg0, sg1, ss0, ss1):
    cid = lax.axis_index("c")
    sid = lax.axis_index("s")
    wid = cid * NS + sid
    S_ = (s0, s1)
    D = (d0, d1)
    T = (t0, t1)
    G = (g0, g1)
    CB = (cb0, cb1)
    NM = (n0, n1)
    RW = (r0, r1)
    SI = (si0, si1)
    SG = (sg0, sg1)
    SS = (ss0, ss1)

    def zrow(r, carry):
        for k in range(HV):
            r0[r, pl.ds(k * LANES, LANES)] = jnp.zeros((LANES,), jnp.float32)
        return carry

    lax.fori_loop(0, SCH, zrow, 0)
    arow = sid * 1000

    @pl.when(sid < 10)
    def _():
        def zcopy(i, carry):
            pltpu.sync_copy(r0.at[pl.ds(0, SCH)],
                            acc.at[pl.ds(arow + i * SCH, SCH)])
            return carry

        lax.fori_loop(0, _NSTRIP, zcopy, 0)
        pltpu.sync_copy(r0.at[pl.ds(0, _TAIL)],
                        acc.at[pl.ds(arow + _NSTRIP * SCH, _TAIL)])

    @pl.when(sid == 10)
    def _():
        pltpu.sync_copy(r0.at[pl.ds(0, 8)], acc.at[pl.ds(N, 8)])

    plsc.subcore_barrier()

    def c_of(i):
        c = wid + i * NW
        return jnp.where(c >= NCH_S, wid, c)

    def is_pad(i):
        return (wid + i * NW) >= NCH_S

    def issue_idx(s, i):
        base = c_of(i) * SCH
        pltpu.async_copy(src_hbm.at[pl.ds(base, SCH)], S_[s], SI[s])
        pltpu.async_copy(dst_hbm.at[pl.ds(base, SCH)], D[s], SI[s])
        pltpu.async_copy(typ_hbm.at[pl.ds(base, SCH)], T[s], SI[s])

    def wait_idx(s):
        pltpu.make_async_copy(src_hbm.at[pl.ds(0, SCH)], S_[s], SI[s]).wait()
        pltpu.make_async_copy(dst_hbm.at[pl.ds(0, SCH)], D[s], SI[s]).wait()
        pltpu.make_async_copy(typ_hbm.at[pl.ds(0, SCH)], T[s], SI[s]).wait()

    def compute_idx(s):
        def vec(v, carry):
            sl = pl.ds(v * LANES, LANES)
            G[s][sl] = T[s][sl] * N + S_[s][sl]
            CB[s][sl] = D[s][sl] * R + T[s][sl]
            return carry

        lax.fori_loop(0, SCH // LANES, vec, 0)

    def issue_gather(s):
        pltpu.async_copy(ytab_hbm.at[G[s]], RW[s], SG[s])
        pltpu.async_copy(inv_hbm.at[CB[s]], NM[s], SG[s])

    def wait_gather(s):
        pltpu.make_async_copy(ytab_hbm.at[G[s]], RW[s], SG[s]).wait()
        pltpu.make_async_copy(inv_hbm.at[CB[s]], NM[s], SG[s]).wait()

    def issue_scat(s):
        pltpu.async_copy(RW[s], acc.at[D[s]], SS[s], add=True)

    def wait_scat(s):
        pltpu.make_async_copy(RW[s], acc.at[D[s]], SS[s]).wait()

    def scale(s, i):
        zv = jnp.zeros((LANES,), jnp.int32)

        def body(e, carry):
            nv = plsc.load_gather(NM[s], [jnp.broadcast_to(e, (LANES,)), zv])
            for k in range(HV):
                sl = pl.ds(k * LANES, LANES)
                RW[s][e, sl] = RW[s][e, sl] * nv
            return carry

        lax.fori_loop(0, SCH, body, 0)

        @pl.when(is_pad(i))
        def _():
            for v in range(SCH // LANES):
                D[s][pl.ds(v * LANES, LANES)] = jnp.full((LANES,), N,
                                                         jnp.int32)

    def half(i, s):
        t = 1 - s
        wait_scat(t)
        issue_idx(t, i + 1)
        wait_gather(s)
        wait_idx(t)
        compute_idx(t)
        issue_gather(t)
        scale(s, i)
        issue_scat(s)

    issue_idx(0, 0)
    wait_idx(0)
    compute_idx(0)
    issue_gather(0)
    issue_idx(1, 1)
    wait_gather(0)
    wait_idx(1)
    compute_idx(1)
    issue_gather(1)
    scale(0, 0)
    issue_scat(0)

    def pair(p, carry):
        half(1 + 2 * p, 1)
        half(2 + 2 * p, 0)
        return carry

    lax.fori_loop(0, (_NITER - 1) // 2, pair, 0)
    wait_gather(1)
    wait_scat(0)

    plsc.subcore_barrier()

    @pl.when(sid < 10)
    def _():
        def dump(i, carry):
            pltpu.sync_copy(acc.at[pl.ds(arow + i * SCH, SCH)], r0)
            pltpu.sync_copy(r0, part_hbm.at[cid, pl.ds(arow + i * SCH, SCH)])
            return carry

        lax.fori_loop(0, _NSTRIP, dump, 0)
        pltpu.sync_copy(acc.at[pl.ds(arow + _NSTRIP * SCH, _TAIL)],
                        r0.at[pl.ds(0, _TAIL)])
        pltpu.sync_copy(r0.at[pl.ds(0, _TAIL)],
                        part_hbm.at[cid, pl.ds(arow + _NSTRIP * SCH, _TAIL)])


def _c_body(relu, p_ref, z_ref, out_ref):
    y = p_ref[0] + p_ref[1] + z_ref[...]
    if relu:
        y = jnp.maximum(y, 0.0)
    out_ref[...] = y


def _c(parts, z, relu):
    return pl.pallas_call(
        functools.partial(_c_body, relu),
        grid=(N // _TM,),
        in_specs=[
            pl.BlockSpec((NC, _TM, H), lambda i: (0, i, 0)),
            pl.BlockSpec((_TM, H), lambda i: (i, 0)),
        ],
        out_specs=pl.BlockSpec((_TM, H), lambda i: (i, 0)),
        out_shape=jax.ShapeDtypeStruct((N, H), jnp.float32),
    )(parts, z)


def _block_diag_weights(w, root):
    wd = jnp.zeros((R, NB, BS, NB, BS), jnp.float32)
    idx = jnp.arange(NB)
    wd = wd.at[:, idx, :, idx, :].set(w.transpose(1, 0, 2, 3))
    wd = wd.reshape(R, H, H)
    return jnp.concatenate([wd, root[None]], axis=0)


def kernel(node_emb, w0, root0, b0, w1, root1, b1, edge_index, edge_type):
    src = edge_index[0]
    dst = edge_index[1]
    ones8 = jnp.ones((CH, 8), jnp.float32)
    zeros8 = jnp.zeros((_CSTRIP, 8), jnp.float32)
    cnt = _p1(dst, edge_type, ones8, zeros8)
    inv = _p2(cnt)

    wall0 = _block_diag_weights(w0, root0)
    wall1 = _block_diag_weights(w1, root1)

    x = node_emb
    for wall, b, relu in ((wall0, b0, True), (wall1, b1, True),
                          (wall1, b1, False)):
        ytab = _t(x, wall, b)
        parts = _s(ytab.reshape((R + 1) * N, H), src, dst, edge_type, inv)
        x = _c(parts, ytab[R].reshape(N, H), relu)
    return x
